# baseline (speedup 1.0000x reference)
import functools

import jax
import jax.numpy as jnp
from jax import lax
from jax.experimental import pallas as pl
from jax.experimental.pallas import tpu as pltpu
from jax.experimental.pallas import tpu_sc as plsc

_NM = 100
_MT = 2048
_B = 4096
_NC, _NS, _L = 2, 16, 16
_NW = _NC * _NS
_BPW = _B // _NW

_PLANE = _NM * _MT
_QM, _QB, _QTG, _QC = 15 * 16 * 4 * 128, 16 * 4 * 128, 4 * 128, 128


def _row_index_prologue(wm_hbm, ts_hbm, tot_hbm, wm_v, ts_v, totg_v, sem,
                        base):
    pltpu.sync_copy(wm_hbm.at[pl.ds(base, _BPW)], wm_v)
    pltpu.sync_copy(ts_hbm.at[pl.ds(base, _BPW)], ts_v)
    pltpu.async_copy(tot_hbm.at[wm_v], totg_v, sem).wait()


def _mk_one_body_gather():
    nf = 45
    mesh = plsc.VectorSubcoreMesh(core_axis_name="c", subcore_axis_name="s",
                                  num_cores=_NC, num_subcores=_NS)
    out_type = jax.ShapeDtypeStruct((_NW * nf * _BPW,), jnp.float32)
    scratch = [
        pltpu.VMEM((45 * _BPW,), jnp.int32),
        pltpu.VMEM((nf * _BPW,), jnp.float32),
        pltpu.SemaphoreType.DMA,
    ]

    @functools.partial(
        pl.kernel, mesh=mesh, out_type=out_type, scratch_types=scratch,
        compiler_params=pltpu.CompilerParams(use_tc_tiling_on_sc=False))
    def k(ixp_hbm, tbl_hbm, out_hbm, ix_v, data_v, sem):
        wid = lax.axis_index("s") * _NC + lax.axis_index("c")
        n = 45 * _BPW
        pltpu.sync_copy(ixp_hbm.at[pl.ds(wid * n, n)], ix_v)
        pltpu.async_copy(tbl_hbm.at[ix_v], data_v, sem).wait()
        pltpu.sync_copy(data_v, out_hbm.at[pl.ds(wid * n, n)])

    return k


def _mk_quat_joint_gather():
    nf = 118
    mesh = plsc.VectorSubcoreMesh(core_axis_name="c", subcore_axis_name="s",
                                  num_cores=_NC, num_subcores=_NS)
    out_type = (jax.ShapeDtypeStruct((_NW * nf * _BPW,), jnp.float32),
                jax.ShapeDtypeStruct((_NW * 45 * _BPW,), jnp.int32))
    scratch = [
        pltpu.VMEM((_BPW,), jnp.int32),
        pltpu.VMEM((_BPW,), jnp.int32),
        pltpu.VMEM((_BPW,), jnp.int32),
        pltpu.VMEM((_BPW,), jnp.int32),
        pltpu.VMEM((_BPW,), jnp.int32),
        pltpu.VMEM((60 * _BPW,), jnp.int32),
        pltpu.VMEM((29 * _BPW,), jnp.int32),
        pltpu.VMEM((45 * _BPW,), jnp.int32),
        pltpu.VMEM((nf * _BPW,), jnp.float32),
        pltpu.SemaphoreType.DMA,
    ]

    @functools.partial(
        pl.kernel, mesh=mesh, out_type=out_type, scratch_types=scratch,
        compiler_params=pltpu.CompilerParams(use_tc_tiling_on_sc=False))
    def k(wm_hbm, ts_hbm, tot_hbm, q_hbm, jp_hbm, jv_hbm, out_hbm, ixp_hbm,
          wm_v, ts_v, totg_v, qb_v, ridx_v, ixq_v, ixj_v, ixp_v, data_v, sem):
        wid = lax.axis_index("s") * _NC + lax.axis_index("c")
        base = wid * _BPW
        _row_index_prologue(wm_hbm, ts_hbm, tot_hbm, wm_v, ts_v, totg_v, sem,
                            base)
        for j8 in range(_BPW // _L):
            sl = pl.ds(j8 * _L, _L)
            wm16 = jnp.minimum(wm_v[sl], _NM - 1)
            ts16 = jnp.minimum(ts_v[sl], totg_v[sl] - 1)
            qb_v[sl] = wm16 * _QM + (ts16 >> 7) * _QTG + (ts16 & 127)
            ridx_v[sl] = wm16 * _MT + ts16

        def qbody(f, _):
            off = (f >> 2) * _QB + (f & 3) * _QC
            for j8 in range(_BPW // _L):
                sl = pl.ds(j8 * _L, _L)
                ixq_v[pl.ds(f * _BPW + j8 * _L, _L)] = qb_v[sl] + off
            return 0
        lax.fori_loop(0, 60, qbody, 0, unroll=False)

        def jbody(f, _):
            for j8 in range(_BPW // _L):
                sl = pl.ds(j8 * _L, _L)
                ixj_v[pl.ds(f * _BPW + j8 * _L, _L)] = ridx_v[sl] + f * _PLANE
            return 0
        lax.fori_loop(0, 29, jbody, 0, unroll=False)

        def pbody(f, _):
            for j8 in range(_BPW // _L):
                sl = pl.ds(j8 * _L, _L)
                ixp_v[pl.ds(f * _BPW + j8 * _L, _L)] = ridx_v[sl] + f * _PLANE
            return 0
        lax.fori_loop(0, 45, pbody, 0, unroll=False)
        pltpu.sync_copy(ixp_v, ixp_hbm.at[pl.ds(wid * 45 * _BPW, 45 * _BPW)])

        copies = [
            pltpu.async_copy(q_hbm.at[ixq_v],
                             data_v.at[pl.ds(0, 60 * _BPW)], sem),
            pltpu.async_copy(jp_hbm.at[ixj_v],
                             data_v.at[pl.ds(60 * _BPW, 29 * _BPW)], sem),
            pltpu.async_copy(jv_hbm.at[ixj_v],
                             data_v.at[pl.ds(89 * _BPW, 29 * _BPW)], sem),
        ]
        for c in copies:
            c.wait()
        pltpu.sync_copy(data_v, out_hbm.at[pl.ds(wid * nf * _BPW, nf * _BPW)])

    return k


def _elu(h):
    return jnp.where(h > 0, h, jnp.exp(jnp.minimum(h, 0.0)) - 1.0)


def _mlp_body(x_ref, w1_ref, b1_ref, w2_ref, b2_ref, w3_ref, b3_ref,
              w4_ref, b4_ref, o_ref):
    h = x_ref[...]
    h = _elu(jnp.dot(h, w1_ref[...], preferred_element_type=jnp.float32)
             + b1_ref[...])
    h = _elu(jnp.dot(h, w2_ref[...], preferred_element_type=jnp.float32)
             + b2_ref[...])
    h = _elu(jnp.dot(h, w3_ref[...], preferred_element_type=jnp.float32)
             + b3_ref[...])
    o_ref[...] = (jnp.dot(h, w4_ref[...], preferred_element_type=jnp.float32)
                  + b4_ref[...])


def _mlp(x, W1, b1, W2, b2, W3, b3, W4, b4):
    bm = 512
    obs = x.shape[1]
    act = W4.shape[1]
    grid = (x.shape[0] // bm,)
    b1, b2, b3, b4 = (b.reshape(1, -1) for b in (b1, b2, b3, b4))

    def _full(w):
        return pl.BlockSpec(w.shape, lambda i: (0,) * w.ndim)

    return pl.pallas_call(
        _mlp_body,
        grid=grid,
        in_specs=[pl.BlockSpec((bm, obs), lambda i: (i, 0)),
                  _full(W1), _full(b1), _full(W2), _full(b2),
                  _full(W3), _full(b3), _full(W4), _full(b4)],
        out_specs=pl.BlockSpec((bm, act), lambda i: (i, 0)),
        out_shape=jax.ShapeDtypeStruct((x.shape[0], act), jnp.float32),
    )(x, W1, b1, W2, b2, W3, b3, W4, b4)


def kernel(x, which_motion, time_step, joint_pos, joint_vel, body_pos_w,
           body_quat_w, body_lin_vel_w, body_ang_vel_w, time_step_totals,
           W1, b1, W2, b2, W3, b3, W4, b4):
    wm = which_motion.astype(jnp.int32).reshape(-1)
    ts = time_step.astype(jnp.int32).reshape(-1)
    totals = time_step_totals.astype(jnp.int32)
    pos_f = jnp.transpose(body_pos_w, (2, 3, 0, 1)).reshape(-1)
    lin_f = jnp.transpose(body_lin_vel_w, (2, 3, 0, 1)).reshape(-1)
    ang_f = jnp.transpose(body_ang_vel_w, (2, 3, 0, 1)).reshape(-1)
    jp_f = jnp.transpose(joint_pos, (2, 0, 1)).reshape(-1)
    jv_f = jnp.transpose(joint_vel, (2, 0, 1)).reshape(-1)
    quat_f = (body_quat_w.reshape(_NM, 16, 128, 15, 4)
              .transpose(0, 3, 1, 4, 2).reshape(-1))

    qj_flat, ixp = _mk_quat_joint_gather()(wm, ts, totals, quat_f, jp_f, jv_f)
    pos_g = _mk_one_body_gather()(ixp, pos_f)
    lin_g = _mk_one_body_gather()(ixp, lin_f)
    ang_g = _mk_one_body_gather()(ixp, ang_f)

    a_qj = qj_flat.reshape(_NW, 118, _BPW)
    a_pos = pos_g.reshape(_NW, 45, _BPW)
    a_lin = lin_g.reshape(_NW, 45, _BPW)
    a_ang = ang_g.reshape(_NW, 45, _BPW)

    def take(arr, lo, n):
        return arr[:, lo:lo + n, :].transpose(0, 2, 1).reshape(_B, n)

    action = _mlp(x, W1, b1, W2, b2, W3, b3, W4, b4)
    return (
        action,
        take(a_qj, 60, 29),
        take(a_qj, 89, 29),
        take(a_pos, 0, 45).reshape(_B, 15, 3),
        take(a_qj, 0, 60).reshape(_B, 15, 4),
        take(a_lin, 0, 45).reshape(_B, 15, 3),
        take(a_ang, 0, 45).reshape(_B, 15, 3),
    )


# speedup vs baseline: 1.0836x; 1.0836x over previous
import functools

import jax
import jax.numpy as jnp
from jax import lax
from jax.experimental import pallas as pl
from jax.experimental.pallas import tpu as pltpu
from jax.experimental.pallas import tpu_sc as plsc

_NM = 100
_MT = 2048
_B = 4096
_NC, _NS, _L = 2, 16, 16
_NW = _NC * _NS
_BPW = _B // _NW

_PLANE = _NM * _MT
_QM, _QB, _QTG, _QC = 15 * 16 * 4 * 128, 16 * 4 * 128, 4 * 128, 128


def _row_index_prologue(wm_hbm, ts_hbm, tot_hbm, wm_v, ts_v, totg_v, sem,
                        base):
    pltpu.sync_copy(wm_hbm.at[pl.ds(base, _BPW)], wm_v)
    pltpu.sync_copy(ts_hbm.at[pl.ds(base, _BPW)], ts_v)
    pltpu.async_copy(tot_hbm.at[wm_v], totg_v, sem).wait()


def _mk_body_gather():
    nf = 135
    mesh = plsc.VectorSubcoreMesh(core_axis_name="c", subcore_axis_name="s",
                                  num_cores=_NC, num_subcores=_NS)
    out_type = jax.ShapeDtypeStruct((_NW * nf * _BPW,), jnp.float32)
    scratch = [
        pltpu.VMEM((45 * _BPW,), jnp.int32),
        pltpu.VMEM((nf * _BPW,), jnp.float32),
        pltpu.SemaphoreType.DMA,
    ]

    @functools.partial(
        pl.kernel, mesh=mesh, out_type=out_type, scratch_types=scratch,
        compiler_params=pltpu.CompilerParams(use_tc_tiling_on_sc=False))
    def k(ixp_hbm, pos_hbm, lin_hbm, ang_hbm, out_hbm,
          ix_v, data_v, sem):
        wid = lax.axis_index("s") * _NC + lax.axis_index("c")
        n = 45 * _BPW
        pltpu.sync_copy(ixp_hbm.at[pl.ds(wid * n, n)], ix_v)
        copies = [
            pltpu.async_copy(pos_hbm.at[ix_v],
                             data_v.at[pl.ds(0, n)], sem),
            pltpu.async_copy(lin_hbm.at[ix_v],
                             data_v.at[pl.ds(n, n)], sem),
            pltpu.async_copy(ang_hbm.at[ix_v],
                             data_v.at[pl.ds(2 * n, n)], sem),
        ]
        for c in copies:
            c.wait()
        pltpu.sync_copy(data_v, out_hbm.at[pl.ds(wid * nf * _BPW, nf * _BPW)])

    return k


def _mk_quat_joint_gather():
    nf = 118
    mesh = plsc.VectorSubcoreMesh(core_axis_name="c", subcore_axis_name="s",
                                  num_cores=_NC, num_subcores=_NS)
    out_type = (jax.ShapeDtypeStruct((_NW * nf * _BPW,), jnp.float32),
                jax.ShapeDtypeStruct((_NW * 45 * _BPW,), jnp.int32))
    scratch = [
        pltpu.VMEM((_BPW,), jnp.int32),
        pltpu.VMEM((_BPW,), jnp.int32),
        pltpu.VMEM((_BPW,), jnp.int32),
        pltpu.VMEM((_BPW,), jnp.int32),
        pltpu.VMEM((_BPW,), jnp.int32),
        pltpu.VMEM((60 * _BPW,), jnp.int32),
        pltpu.VMEM((29 * _BPW,), jnp.int32),
        pltpu.VMEM((45 * _BPW,), jnp.int32),
        pltpu.VMEM((nf * _BPW,), jnp.float32),
        pltpu.SemaphoreType.DMA,
    ]

    @functools.partial(
        pl.kernel, mesh=mesh, out_type=out_type, scratch_types=scratch,
        compiler_params=pltpu.CompilerParams(use_tc_tiling_on_sc=False))
    def k(wm_hbm, ts_hbm, tot_hbm, q_hbm, jp_hbm, jv_hbm, out_hbm, ixp_hbm,
          wm_v, ts_v, totg_v, qb_v, ridx_v, ixq_v, ixj_v, ixp_v, data_v, sem):
        wid = lax.axis_index("s") * _NC + lax.axis_index("c")
        base = wid * _BPW
        _row_index_prologue(wm_hbm, ts_hbm, tot_hbm, wm_v, ts_v, totg_v, sem,
                            base)
        for j8 in range(_BPW // _L):
            sl = pl.ds(j8 * _L, _L)
            wm16 = jnp.minimum(wm_v[sl], _NM - 1)
            ts16 = jnp.minimum(ts_v[sl], totg_v[sl] - 1)
            qb_v[sl] = wm16 * _QM + (ts16 >> 7) * _QTG + (ts16 & 127)
            ridx_v[sl] = wm16 * _MT + ts16

        def qbody(f, _):
            off = (f >> 2) * _QB + (f & 3) * _QC
            for j8 in range(_BPW // _L):
                sl = pl.ds(j8 * _L, _L)
                ixq_v[pl.ds(f * _BPW + j8 * _L, _L)] = qb_v[sl] + off
            return 0
        lax.fori_loop(0, 60, qbody, 0, unroll=False)

        def jbody(f, _):
            for j8 in range(_BPW // _L):
                sl = pl.ds(j8 * _L, _L)
                ixj_v[pl.ds(f * _BPW + j8 * _L, _L)] = ridx_v[sl] + f * _PLANE
            return 0
        lax.fori_loop(0, 29, jbody, 0, unroll=False)

        def pbody(f, _):
            for j8 in range(_BPW // _L):
                sl = pl.ds(j8 * _L, _L)
                ixp_v[pl.ds(f * _BPW + j8 * _L, _L)] = ridx_v[sl] + f * _PLANE
            return 0
        lax.fori_loop(0, 45, pbody, 0, unroll=False)
        pltpu.sync_copy(ixp_v, ixp_hbm.at[pl.ds(wid * 45 * _BPW, 45 * _BPW)])

        copies = [
            pltpu.async_copy(q_hbm.at[ixq_v],
                             data_v.at[pl.ds(0, 60 * _BPW)], sem),
            pltpu.async_copy(jp_hbm.at[ixj_v],
                             data_v.at[pl.ds(60 * _BPW, 29 * _BPW)], sem),
            pltpu.async_copy(jv_hbm.at[ixj_v],
                             data_v.at[pl.ds(89 * _BPW, 29 * _BPW)], sem),
        ]
        for c in copies:
            c.wait()
        pltpu.sync_copy(data_v, out_hbm.at[pl.ds(wid * nf * _BPW, nf * _BPW)])

    return k


def _elu(h):
    return jnp.where(h > 0, h, jnp.exp(jnp.minimum(h, 0.0)) - 1.0)


def _mlp_body(x_ref, w1_ref, b1_ref, w2_ref, b2_ref, w3_ref, b3_ref,
              w4_ref, b4_ref, o_ref):
    h = x_ref[...]
    h = _elu(jnp.dot(h, w1_ref[...], preferred_element_type=jnp.float32)
             + b1_ref[...])
    h = _elu(jnp.dot(h, w2_ref[...], preferred_element_type=jnp.float32)
             + b2_ref[...])
    h = _elu(jnp.dot(h, w3_ref[...], preferred_element_type=jnp.float32)
             + b3_ref[...])
    o_ref[...] = (jnp.dot(h, w4_ref[...], preferred_element_type=jnp.float32)
                  + b4_ref[...])


def _mlp(x, W1, b1, W2, b2, W3, b3, W4, b4):
    bm = 512
    obs = x.shape[1]
    act = W4.shape[1]
    grid = (x.shape[0] // bm,)
    b1, b2, b3, b4 = (b.reshape(1, -1) for b in (b1, b2, b3, b4))

    def _full(w):
        return pl.BlockSpec(w.shape, lambda i: (0,) * w.ndim)

    return pl.pallas_call(
        _mlp_body,
        grid=grid,
        in_specs=[pl.BlockSpec((bm, obs), lambda i: (i, 0)),
                  _full(W1), _full(b1), _full(W2), _full(b2),
                  _full(W3), _full(b3), _full(W4), _full(b4)],
        out_specs=pl.BlockSpec((bm, act), lambda i: (i, 0)),
        out_shape=jax.ShapeDtypeStruct((x.shape[0], act), jnp.float32),
    )(x, W1, b1, W2, b2, W3, b3, W4, b4)


def kernel(x, which_motion, time_step, joint_pos, joint_vel, body_pos_w,
           body_quat_w, body_lin_vel_w, body_ang_vel_w, time_step_totals,
           W1, b1, W2, b2, W3, b3, W4, b4):
    wm = which_motion.astype(jnp.int32).reshape(-1)
    ts = time_step.astype(jnp.int32).reshape(-1)
    totals = time_step_totals.astype(jnp.int32)
    pos_f = jnp.transpose(body_pos_w, (2, 3, 0, 1)).reshape(-1)
    lin_f = jnp.transpose(body_lin_vel_w, (2, 3, 0, 1)).reshape(-1)
    ang_f = jnp.transpose(body_ang_vel_w, (2, 3, 0, 1)).reshape(-1)
    jp_f = jnp.transpose(joint_pos, (2, 0, 1)).reshape(-1)
    jv_f = jnp.transpose(joint_vel, (2, 0, 1)).reshape(-1)
    quat_f = (body_quat_w.reshape(_NM, 16, 128, 15, 4)
              .transpose(0, 3, 1, 4, 2).reshape(-1))

    qj_flat, ixp = _mk_quat_joint_gather()(wm, ts, totals, quat_f, jp_f, jv_f)
    body_flat = _mk_body_gather()(ixp, pos_f, lin_f, ang_f)

    a_qj = qj_flat.reshape(_NW, 118, _BPW)
    a_body = body_flat.reshape(_NW, 135, _BPW)

    def take(arr, lo, n):
        return arr[:, lo:lo + n, :].transpose(0, 2, 1).reshape(_B, n)

    action = _mlp(x, W1, b1, W2, b2, W3, b3, W4, b4)
    return (
        action,
        take(a_qj, 60, 29),
        take(a_qj, 89, 29),
        take(a_body, 0, 45).reshape(_B, 15, 3),
        take(a_qj, 0, 60).reshape(_B, 15, 4),
        take(a_body, 45, 45).reshape(_B, 15, 3),
        take(a_body, 90, 45).reshape(_B, 15, 3),
    )
